# baseline (speedup 1.0000x reference)
import functools

import jax
import jax.numpy as jnp
from jax import lax
from jax.experimental import pallas as pl
from jax.experimental.pallas import tpu as pltpu
from jax.experimental.pallas import tpu_sc as plsc

_D = 64
_L = 16
_NSEG = 3
_NC, _NS = 2, 16
_NW = _NC * _NS
_BTW = 4


def _sc_body(x_hbm, ids_hbm, tab_hbm, out_hbm,
             xbuf, idsbuf, tabv, tabr,
             sem_in0, sem_in1, sem_out0, sem_out1):
    steps = x_hbm.shape[0]
    npairs = steps // 2
    wid = lax.axis_index("s") * _NC + lax.axis_index("c")
    bt0 = wid * _BTW

    pltpu.sync_copy(tab_hbm, tabv)
    for seg in range(_NSEG):
        for j in range(_D // _L):
            row = tabv[pl.ds(seg * _D + j * _L, _L)]
            for l in range(_L):
                d = j * _L + l
                tabr[pl.ds(_L * (4 * d + seg), _L)] = jnp.full(
                    (_L,), row[l], dtype=jnp.float32)
    iota = lax.iota(jnp.int32, _L)

    sems_in = (sem_in0, sem_in1)
    sems_out = (sem_out0, sem_out1)

    def start_in(g, slot):
        pltpu.async_copy(x_hbm.at[g, :, pl.ds(bt0, _BTW)], xbuf.at[slot],
                         sems_in[slot])
        pltpu.async_copy(ids_hbm.at[g // 8, pl.ds(bt0, _BTW)],
                         idsbuf.at[slot], sems_in[slot])

    def wait_in(slot):
        pltpu.make_async_copy(x_hbm.at[0, :, pl.ds(bt0, _BTW)],
                              xbuf.at[slot], sems_in[slot]).wait()
        pltpu.make_async_copy(ids_hbm.at[0, pl.ds(bt0, _BTW)],
                              idsbuf.at[slot], sems_in[slot]).wait()

    def start_out(g, slot):
        pltpu.async_copy(xbuf.at[slot], out_hbm.at[g, :, pl.ds(bt0, _BTW)],
                         sems_out[slot])

    def wait_out(slot):
        pltpu.make_async_copy(xbuf.at[slot],
                              out_hbm.at[0, :, pl.ds(bt0, _BTW)],
                              sems_out[slot]).wait()

    def compute(g, slot):
        sr = g % 8

        @plsc.parallel_loop(0, _BTW * 8, step=1)
        def _(g0):
            bt = g0 // 8
            blg = g0 % 8
            fs = pl.ds(blg * _L, _L)
            ids16 = idsbuf[slot, bt, sr, fs]
            grpbase = iota + ids16 * _L
            for d in range(_D):
                dt, dr = d // 8, d % 8
                row = plsc.load_gather(tabr, [grpbase + d * 4 * _L])
                xbuf[slot, dt, bt, dr, fs] = (
                    xbuf[slot, dt, bt, dr, fs] + row)

    start_in(0, 0)
    start_in(1, 1)

    def pair_body(gg, carry):
        g0 = 2 * gg
        wait_in(0)
        compute(g0, 0)
        start_out(g0, 0)
        wait_in(1)
        compute(g0 + 1, 1)
        start_out(g0 + 1, 1)

        @pl.when(gg + 1 < npairs)
        def _():
            wait_out(0)
            start_in(g0 + 2, 0)
            wait_out(1)
            start_in(g0 + 3, 1)

        return carry

    lax.fori_loop(0, npairs, pair_body, 0)
    wait_out(0)
    wait_out(1)


def kernel(x, segment_ids, segment_table):
    b, s, d = x.shape
    x_t = (x.transpose(1, 2, 0)
           .reshape(s, d // 8, 8, b // 128, 128)
           .transpose(0, 1, 3, 2, 4))
    ids_t = (segment_ids.astype(jnp.int32).transpose(1, 0)
             .reshape(s // 8, 8, b // 128, 128)
             .transpose(0, 2, 1, 3))
    fn = pl.kernel(
        _sc_body,
        out_type=jax.ShapeDtypeStruct(x_t.shape, jnp.float32),
        mesh=plsc.VectorSubcoreMesh(core_axis_name="c", subcore_axis_name="s",
                                    num_cores=_NC, num_subcores=_NS),
        compiler_params=pltpu.CompilerParams(needs_layout_passes=False,
                                             use_tc_tiling_on_sc=False),
        scratch_types=[
            pltpu.VMEM((2, 8, _BTW, 8, 128), jnp.float32),
            pltpu.VMEM((2, _BTW, 8, 128), jnp.int32),
            pltpu.VMEM((_NSEG * _D,), jnp.float32),
            pltpu.VMEM((_D * 4 * _L,), jnp.float32),
            pltpu.SemaphoreType.DMA,
            pltpu.SemaphoreType.DMA,
            pltpu.SemaphoreType.DMA,
            pltpu.SemaphoreType.DMA,
        ],
    )
    out_t = fn(x_t, ids_t, segment_table.reshape(-1))
    return (out_t.transpose(0, 1, 3, 2, 4)
            .reshape(s, d, b)
            .transpose(2, 0, 1))


# speedup vs baseline: 1.0684x; 1.0684x over previous
import functools

import jax
import jax.numpy as jnp
from jax import lax
from jax.experimental import pallas as pl
from jax.experimental.pallas import tpu as pltpu
from jax.experimental.pallas import tpu_sc as plsc

_D = 64
_L = 16
_NSEG = 3
_NC, _NS = 2, 16
_NW = _NC * _NS
_BTW = 4


def _sc_body(x_hbm, ids_hbm, tab_hbm, out_hbm,
             xbuf, idsbuf, tabv, tabr,
             sem_in0, sem_in1, sem_out0, sem_out1):
    steps = x_hbm.shape[0]
    npairs = steps // 2
    wid = lax.axis_index("s") * _NC + lax.axis_index("c")
    bt0 = wid * _BTW

    pltpu.sync_copy(tab_hbm, tabv)
    for seg in range(_NSEG):
        for j in range(_D // _L):
            row = tabv[pl.ds(seg * _D + j * _L, _L)]
            for l in range(_L):
                d = j * _L + l
                tabr[pl.ds(_L * (4 * d + seg), _L)] = jnp.full(
                    (_L,), row[l], dtype=jnp.float32)
    iota = lax.iota(jnp.int32, _L)

    sems_in = (sem_in0, sem_in1)
    sems_out = (sem_out0, sem_out1)

    def start_in(g, slot):
        pltpu.async_copy(x_hbm.at[g, :, pl.ds(bt0, _BTW)], xbuf.at[slot],
                         sems_in[slot])
        pltpu.async_copy(ids_hbm.at[g // 8, pl.ds(bt0, _BTW)],
                         idsbuf.at[slot], sems_in[slot])

    def wait_in(slot):
        pltpu.make_async_copy(x_hbm.at[0, :, pl.ds(bt0, _BTW)],
                              xbuf.at[slot], sems_in[slot]).wait()
        pltpu.make_async_copy(ids_hbm.at[0, pl.ds(bt0, _BTW)],
                              idsbuf.at[slot], sems_in[slot]).wait()

    def start_out(g, slot):
        pltpu.async_copy(xbuf.at[slot], out_hbm.at[g, :, pl.ds(bt0, _BTW)],
                         sems_out[slot])

    def wait_out(slot):
        pltpu.make_async_copy(xbuf.at[slot],
                              out_hbm.at[0, :, pl.ds(bt0, _BTW)],
                              sems_out[slot]).wait()

    def compute(g, slot):
        sr = g % 8

        @plsc.parallel_loop(0, _BTW * 8, step=1)
        def _(g0):
            bt = g0 // 8
            blg = g0 % 8
            fs = pl.ds(blg * _L, _L)
            ids16 = idsbuf[slot, bt, sr, fs]
            grpbase = iota + ids16 * _L
            for d in range(_D):
                dt, dr = d // 8, d % 8
                row = plsc.load_gather(tabr, [grpbase + d * 4 * _L])
                plsc.addupdate(xbuf.at[slot, dt, bt, dr, fs], row)

    start_in(0, 0)
    start_in(1, 1)

    def pair_body(gg, carry):
        g0 = 2 * gg
        wait_in(0)
        compute(g0, 0)
        start_out(g0, 0)
        wait_in(1)
        compute(g0 + 1, 1)
        start_out(g0 + 1, 1)

        @pl.when(gg + 1 < npairs)
        def _():
            wait_out(0)
            start_in(g0 + 2, 0)
            wait_out(1)
            start_in(g0 + 3, 1)

        return carry

    lax.fori_loop(0, npairs, pair_body, 0)
    wait_out(0)
    wait_out(1)


def kernel(x, segment_ids, segment_table):
    b, s, d = x.shape
    x_t = (x.transpose(1, 2, 0)
           .reshape(s, d // 8, 8, b // 128, 128)
           .transpose(0, 1, 3, 2, 4))
    ids_t = (segment_ids.astype(jnp.int32).transpose(1, 0)
             .reshape(s // 8, 8, b // 128, 128)
             .transpose(0, 2, 1, 3))
    fn = pl.kernel(
        _sc_body,
        out_type=jax.ShapeDtypeStruct(x_t.shape, jnp.float32),
        mesh=plsc.VectorSubcoreMesh(core_axis_name="c", subcore_axis_name="s",
                                    num_cores=_NC, num_subcores=_NS),
        compiler_params=pltpu.CompilerParams(needs_layout_passes=False,
                                             use_tc_tiling_on_sc=False),
        scratch_types=[
            pltpu.VMEM((2, 8, _BTW, 8, 128), jnp.float32),
            pltpu.VMEM((2, _BTW, 8, 128), jnp.int32),
            pltpu.VMEM((_NSEG * _D,), jnp.float32),
            pltpu.VMEM((_D * 4 * _L,), jnp.float32),
            pltpu.SemaphoreType.DMA,
            pltpu.SemaphoreType.DMA,
            pltpu.SemaphoreType.DMA,
            pltpu.SemaphoreType.DMA,
        ],
    )
    out_t = fn(x_t, ids_t, segment_table.reshape(-1))
    return (out_t.transpose(0, 1, 3, 2, 4)
            .reshape(s, d, b)
            .transpose(2, 0, 1))


# speedup vs baseline: 1.3821x; 1.2936x over previous
import functools

import jax
import jax.numpy as jnp
from jax import lax
from jax.experimental import pallas as pl
from jax.experimental.pallas import tpu as pltpu
from jax.experimental.pallas import tpu_sc as plsc

_D = 64
_L = 16
_NSEG = 3
_NC, _NS = 2, 16
_NW = _NC * _NS
_BTW = 4


def _sc_body(x_hbm, ids_hbm, tab_hbm, out_hbm,
             xbuf, idsbuf, tabv, tabr,
             sem_in0, sem_in1, sem_in2, sem_out0, sem_out1, sem_out2):
    steps = x_hbm.shape[0]
    ntrip = (steps - 2) // 3
    wid = lax.axis_index("s") * _NC + lax.axis_index("c")
    bt0 = wid * _BTW

    pltpu.sync_copy(tab_hbm, tabv)
    for seg in range(_NSEG):
        for j in range(_D // _L):
            row = tabv[pl.ds(seg * _D + j * _L, _L)]
            for l in range(_L):
                d = j * _L + l
                tabr[pl.ds(_L * (4 * d + seg), _L)] = jnp.full(
                    (_L,), row[l], dtype=jnp.float32)
    iota = lax.iota(jnp.int32, _L)

    sems_in = (sem_in0, sem_in1, sem_in2)
    sems_out = (sem_out0, sem_out1, sem_out2)

    def start_in(g, slot):
        pltpu.async_copy(x_hbm.at[g, :, pl.ds(bt0, _BTW)], xbuf.at[slot],
                         sems_in[slot])
        pltpu.async_copy(ids_hbm.at[g // 8, pl.ds(bt0, _BTW)],
                         idsbuf.at[slot], sems_in[slot])

    def wait_in(slot):
        pltpu.make_async_copy(x_hbm.at[0, :, pl.ds(bt0, _BTW)],
                              xbuf.at[slot], sems_in[slot]).wait()
        pltpu.make_async_copy(ids_hbm.at[0, pl.ds(bt0, _BTW)],
                              idsbuf.at[slot], sems_in[slot]).wait()

    def start_out(g, slot):
        pltpu.async_copy(xbuf.at[slot], out_hbm.at[g, :, pl.ds(bt0, _BTW)],
                         sems_out[slot])

    def wait_out(slot):
        pltpu.make_async_copy(xbuf.at[slot],
                              out_hbm.at[0, :, pl.ds(bt0, _BTW)],
                              sems_out[slot]).wait()

    def compute(g, slot):
        sr = g % 8

        @plsc.parallel_loop(0, _BTW * 8, step=1)
        def _(g0):
            bt = g0 // 8
            blg = g0 % 8
            fs = pl.ds(blg * _L, _L)
            ids16 = idsbuf[slot, bt, sr, fs]
            grpbase = iota + ids16 * _L
            for d in range(_D):
                dt, dr = d // 8, d % 8
                row = plsc.load_gather(tabr, [grpbase + d * 4 * _L])
                plsc.addupdate(xbuf.at[slot, dt, bt, dr, fs], row)

    start_in(0, 0)
    start_in(1, 1)

    def trip_body(gg, carry):
        for k in range(3):
            g = 3 * gg + k
            wait_in(k)
            compute(g, k)
            start_out(g, k)
            kp = (k + 2) % 3
            @pl.when(g + 2 < steps)
            def _():
                @pl.when(g >= 1)
                def _():
                    wait_out(kp)
                start_in(g + 2, kp)

        return carry

    lax.fori_loop(0, ntrip, trip_body, 0)
    ge = 3 * ntrip
    wait_in(0)
    compute(ge, 0)
    start_out(ge, 0)
    wait_in(1)
    compute(ge + 1, 1)
    start_out(ge + 1, 1)
    wait_out(2)
    wait_out(0)
    wait_out(1)


def kernel(x, segment_ids, segment_table):
    b, s, d = x.shape
    x_t = (x.transpose(1, 2, 0)
           .reshape(s, d // 8, 8, b // 128, 128)
           .transpose(0, 1, 3, 2, 4))
    ids_t = (segment_ids.astype(jnp.int32).transpose(1, 0)
             .reshape(s // 8, 8, b // 128, 128)
             .transpose(0, 2, 1, 3))
    fn = pl.kernel(
        _sc_body,
        out_type=jax.ShapeDtypeStruct(x_t.shape, jnp.float32),
        mesh=plsc.VectorSubcoreMesh(core_axis_name="c", subcore_axis_name="s",
                                    num_cores=_NC, num_subcores=_NS),
        compiler_params=pltpu.CompilerParams(needs_layout_passes=False,
                                             use_tc_tiling_on_sc=False),
        scratch_types=[
            pltpu.VMEM((3, 8, _BTW, 8, 128), jnp.float32),
            pltpu.VMEM((3, _BTW, 8, 128), jnp.int32),
            pltpu.VMEM((_NSEG * _D,), jnp.float32),
            pltpu.VMEM((_D * 4 * _L,), jnp.float32),
            pltpu.SemaphoreType.DMA,
            pltpu.SemaphoreType.DMA,
            pltpu.SemaphoreType.DMA,
            pltpu.SemaphoreType.DMA,
            pltpu.SemaphoreType.DMA,
            pltpu.SemaphoreType.DMA,
        ],
    )
    out_t = fn(x_t, ids_t, segment_table.reshape(-1))
    return (out_t.transpose(0, 1, 3, 2, 4)
            .reshape(s, d, b)
            .transpose(2, 0, 1))
